# Initial kernel scaffold; baseline (speedup 1.0000x reference)
import jax
import jax.numpy as jnp
from jax.experimental import pallas as pl


def kernel(inputs, embedding):
    raise NotImplementedError("write your pallas kernel here")


# speedup vs baseline: 1.4416x; 1.4416x over previous
import functools

import jax
import jax.numpy as jnp
from jax import lax
from jax.experimental import pallas as pl
from jax.experimental.pallas import tpu as pltpu
from jax.experimental.pallas import tpu_sc as plsc

_NC = 2
_NS = 16
_NW = _NC * _NS


@functools.partial(jax.jit, static_argnums=(2, 3, 4))
def _sc_gather(embedding, idx3, n_ch, ch, d):
  mesh = plsc.VectorSubcoreMesh(core_axis_name="c", subcore_axis_name="s")

  @functools.partial(
      pl.kernel,
      out_type=jax.ShapeDtypeStruct((_NW, n_ch, ch, d), jnp.float32),
      mesh=mesh,
      scratch_types=[
          pltpu.VMEM((n_ch, ch), jnp.int32),
          pltpu.VMEM((ch, d), jnp.float32),
          pltpu.SemaphoreType.DMA,
      ],
      compiler_params=pltpu.CompilerParams(use_tc_tiling_on_sc=False),
  )
  def k(table_hbm, idx_hbm, out_hbm, idx_v, rows_v, sem):
    wid = lax.axis_index("s") * _NC + lax.axis_index("c")
    pltpu.sync_copy(idx_hbm.at[wid], idx_v)

    def body(j, carry):
      pltpu.async_copy(table_hbm.at[idx_v.at[j]], rows_v, sem).wait()
      pltpu.sync_copy(rows_v, out_hbm.at[wid, j])
      return carry

    lax.fori_loop(0, n_ch, body, 0)

  return k(embedding, idx3)


def kernel(inputs, embedding):
  b, f = inputs.shape
  d = embedding.shape[1]
  total = b * f
  per_w = total // _NW
  ch = 128
  n_ch = per_w // ch
  idx3 = inputs.reshape(_NW, n_ch, ch).astype(jnp.int32)
  out = _sc_gather(embedding, idx3, n_ch, ch, d)
  return out.reshape(b, f, d)


# speedup vs baseline: 1.5659x; 1.0863x over previous
import functools

import jax
import jax.numpy as jnp
from jax import lax
from jax.experimental import pallas as pl
from jax.experimental.pallas import tpu as pltpu
from jax.experimental.pallas import tpu_sc as plsc

_NC = 2
_NS = 16
_NW = _NC * _NS

_CH = 104
_G = 8
_NCH = 128
_NGROUPS = _NCH // _G


@functools.partial(jax.jit, static_argnums=(2,))
def _sc_gather(embedding, idx3, d):
  rows_per_group = _G * _CH
  mesh = plsc.VectorSubcoreMesh(core_axis_name="c", subcore_axis_name="s")

  @functools.partial(
      pl.kernel,
      out_type=jax.ShapeDtypeStruct((_NW, _NGROUPS, rows_per_group, d),
                                    jnp.float32),
      mesh=mesh,
      scratch_types=[
          pltpu.VMEM((_NCH, _CH), jnp.int32),
          pltpu.VMEM((2, rows_per_group, d), jnp.float32),
          pltpu.SemaphoreType.DMA,
          pltpu.SemaphoreType.DMA,
          pltpu.SemaphoreType.DMA,
          pltpu.SemaphoreType.DMA,
      ],
      compiler_params=pltpu.CompilerParams(use_tc_tiling_on_sc=False),
  )
  def k(table_hbm, idx_hbm, out_hbm, idx_v, rows_v, sem0, sem1, wsem0, wsem1):
    wid = lax.axis_index("s") * _NC + lax.axis_index("c")
    pltpu.sync_copy(idx_hbm.at[wid], idx_v)

    def fire_group(g, p, sem):
      for b in range(_G):
        pltpu.async_copy(
            table_hbm.at[idx_v.at[g * _G + b]],
            rows_v.at[p, pl.ds(b * _CH, _CH)],
            sem,
        )

    def drain_group(p, sem):
      pltpu.make_async_copy(out_hbm.at[wid, 0], rows_v.at[p], sem).wait()

    fire_group(0, 0, sem0)
    fire_group(1, 1, sem1)

    def body(i, carry):
      g = 2 * i
      drain_group(0, sem0)
      w0 = pltpu.async_copy(rows_v.at[0], out_hbm.at[wid, g], wsem0)
      drain_group(1, sem1)
      w1 = pltpu.async_copy(rows_v.at[1], out_hbm.at[wid, g + 1], wsem1)
      w0.wait()
      fire_group(g + 2, 0, sem0)
      w1.wait()
      fire_group(g + 3, 1, sem1)
      return carry

    lax.fori_loop(0, _NGROUPS // 2 - 1, body, 0)

    drain_group(0, sem0)
    pltpu.async_copy(rows_v.at[0], out_hbm.at[wid, _NGROUPS - 2], wsem0)
    drain_group(1, sem1)
    pltpu.async_copy(rows_v.at[1], out_hbm.at[wid, _NGROUPS - 1], wsem1)
    pltpu.make_async_copy(rows_v.at[0], out_hbm.at[wid, 0], wsem0).wait()
    pltpu.make_async_copy(rows_v.at[1], out_hbm.at[wid, 0], wsem1).wait()

  return k(embedding, idx3)


def kernel(inputs, embedding):
  b, f = inputs.shape
  d = embedding.shape[1]
  idx3 = inputs.reshape(_NW, _NCH, _CH).astype(jnp.int32)
  out = _sc_gather(embedding, idx3, d)
  return out.reshape(b, f, d)


# speedup vs baseline: 1.6587x; 1.0593x over previous
import functools

import jax
import jax.numpy as jnp
from jax import lax
from jax.experimental import pallas as pl
from jax.experimental.pallas import tpu as pltpu
from jax.experimental.pallas import tpu_sc as plsc

_NC = 2
_NS = 16
_NW = _NC * _NS

_CH = 128
_NFIELD = 26
_BATCH = 16384
_BW = _BATCH // _NW
_NCHF = _BW // _CH


@functools.partial(jax.jit, static_argnums=(2,))
def _sc_gather(embedding, idx_t, d):
  mesh = plsc.VectorSubcoreMesh(core_axis_name="c", subcore_axis_name="s")

  @functools.partial(
      pl.kernel,
      out_type=jax.ShapeDtypeStruct((_NFIELD, _BATCH, d), jnp.float32),
      mesh=mesh,
      scratch_types=[
          pltpu.VMEM((_NFIELD, _BW), jnp.int32),
          pltpu.VMEM((2, _BW, d), jnp.float32),
          pltpu.SemaphoreType.DMA,
          pltpu.SemaphoreType.DMA,
          pltpu.SemaphoreType.DMA,
          pltpu.SemaphoreType.DMA,
      ],
      compiler_params=pltpu.CompilerParams(use_tc_tiling_on_sc=False),
  )
  def k(table_hbm, idx_hbm, out_hbm, idx_v, rows_v, sem0, sem1, wsem0, wsem1):
    wid = lax.axis_index("s") * _NC + lax.axis_index("c")
    base = wid * _BW
    pltpu.sync_copy(idx_hbm.at[:, pl.ds(base, _BW)], idx_v)

    def fire_field(f, p, sem):
      for c in range(_NCHF):
        pltpu.async_copy(
            table_hbm.at[idx_v.at[f, pl.ds(c * _CH, _CH)]],
            rows_v.at[p, pl.ds(c * _CH, _CH)],
            sem,
        )

    def drain_field(p, sem):
      pltpu.make_async_copy(
          out_hbm.at[0, pl.ds(base, _BW)], rows_v.at[p], sem).wait()

    fire_field(0, 0, sem0)
    fire_field(1, 1, sem1)

    def body(i, carry):
      f = 2 * i
      drain_field(0, sem0)
      w0 = pltpu.async_copy(rows_v.at[0], out_hbm.at[f, pl.ds(base, _BW)],
                            wsem0)
      drain_field(1, sem1)
      w1 = pltpu.async_copy(rows_v.at[1], out_hbm.at[f + 1, pl.ds(base, _BW)],
                            wsem1)
      w0.wait()
      fire_field(f + 2, 0, sem0)
      w1.wait()
      fire_field(f + 3, 1, sem1)
      return carry

    lax.fori_loop(0, _NFIELD // 2 - 1, body, 0)

    drain_field(0, sem0)
    pltpu.async_copy(rows_v.at[0], out_hbm.at[_NFIELD - 2, pl.ds(base, _BW)],
                     wsem0)
    drain_field(1, sem1)
    pltpu.async_copy(rows_v.at[1], out_hbm.at[_NFIELD - 1, pl.ds(base, _BW)],
                     wsem1)
    pltpu.make_async_copy(rows_v.at[0], out_hbm.at[0, pl.ds(base, _BW)],
                          wsem0).wait()
    pltpu.make_async_copy(rows_v.at[1], out_hbm.at[0, pl.ds(base, _BW)],
                          wsem1).wait()

  return k(embedding, idx_t)


def kernel(inputs, embedding):
  b, f = inputs.shape
  d = embedding.shape[1]
  idx_t = inputs.T.astype(jnp.int32)
  out = _sc_gather(embedding, idx_t, d)
  return out.transpose(1, 0, 2)
